# Initial kernel scaffold; baseline (speedup 1.0000x reference)
import jax
import jax.numpy as jnp
from jax.experimental import pallas as pl


def kernel(x, edge_index, weight, conv_bias, ntype_bias):
    raise NotImplementedError("write your pallas kernel here")


# speedup vs baseline: 5.2991x; 5.2991x over previous
import jax
import jax.numpy as jnp
from jax import lax
from jax.experimental import pallas as pl
from jax.experimental.pallas import tpu as pltpu
from jax.experimental.pallas import tpu_sc as plsc

N = 10000
D = 128
R = 4
E = 160000
HD = 64
DP = 80
NSUB = 16
NCORE = 2
EPT = E // NSUB
CH = 125
NCH = EPT // CH
NPAD = 10240
SLAB = 128
ZROWS = (NPAD // NSUB) // SLAB


def _sc_body(xp_hbm, src_hbm, dst_hbm, zeros_hbm, out_hbm,
             src_loc, dst_loc, rows, zbuf, agg_sh, sem):
    c = lax.axis_index("c")
    s = lax.axis_index("s")
    pltpu.sync_copy(zeros_hbm, zbuf)
    for r in range(R):
        pltpu.sync_copy(src_hbm.at[r, s], src_loc)
        pltpu.sync_copy(dst_hbm.at[r, s], dst_loc)
        for z in range(ZROWS):
            pltpu.sync_copy(zbuf,
                            agg_sh.at[pl.ds((s * ZROWS + z) * SLAB, SLAB)])
        plsc.subcore_barrier()

        def chunk(j, carry):
            pltpu.async_copy(xp_hbm.at[c].at[src_loc.at[j]], rows, sem).wait()
            pltpu.sync_copy(rows, agg_sh.at[dst_loc.at[j]], add=True)
            return carry

        lax.fori_loop(0, NCH, chunk, 0)
        plsc.subcore_barrier()
        for z in range(ZROWS):
            base = (s * ZROWS + z) * SLAB
            pltpu.sync_copy(agg_sh.at[pl.ds(base, SLAB)],
                            out_hbm.at[c, r, pl.ds(base, SLAB)])


def _sc_aggregate(xp, src, dst, zeros):
    mesh = plsc.VectorSubcoreMesh(core_axis_name="c", subcore_axis_name="s")
    return pl.kernel(
        _sc_body,
        out_type=jax.ShapeDtypeStruct((NCORE, R, NPAD, DP), jnp.float32),
        mesh=mesh,
        scratch_types=[
            pltpu.VMEM((NCH, CH), jnp.int32),
            pltpu.VMEM((NCH, CH), jnp.int32),
            pltpu.VMEM((CH, DP), jnp.float32),
            pltpu.VMEM((SLAB, DP), jnp.float32),
            pltpu.VMEM_SHARED((NPAD, DP), jnp.float32),
            pltpu.SemaphoreType.DMA,
        ],
        compiler_params=pltpu.CompilerParams(use_tc_tiling_on_sc=False),
    )(xp, src, dst, zeros)


BN = 1000


def _tc_body(agg_ref, w_ref, cb_ref, nb_ref, o_ref):
    acc = jnp.zeros((BN, D), jnp.float32)
    for r in range(R):
        deg = jnp.maximum(agg_ref[0, r, :, HD:HD + 1], 1.0)
        a0 = agg_ref[0, r, :, :HD] / deg
        a1 = agg_ref[1, r, :, :HD] / deg
        acc = acc + jnp.dot(a0, w_ref[r, :HD, :],
                            preferred_element_type=jnp.float32)
        acc = acc + jnp.dot(a1, w_ref[r, HD:, :],
                            preferred_element_type=jnp.float32)
        acc = acc + cb_ref[r, :][None, :]
    h = acc * (1.0 / R)
    h = h * jax.nn.sigmoid(h)
    o_ref[...] = h + nb_ref[0, :][None, :]


def _tc_finish(agg, weight, conv_bias, ntype_bias):
    grid = (N // BN,)
    return pl.pallas_call(
        _tc_body,
        grid=grid,
        in_specs=[
            pl.BlockSpec((NCORE, R, BN, DP), lambda i: (0, 0, i, 0)),
            pl.BlockSpec((R, D, D), lambda i: (0, 0, 0)),
            pl.BlockSpec((R, D), lambda i: (0, 0)),
            pl.BlockSpec((1, D), lambda i: (0, 0)),
        ],
        out_specs=pl.BlockSpec((BN, D), lambda i: (i, 0)),
        out_shape=jax.ShapeDtypeStruct((N, D), jnp.float32),
    )(agg, weight, conv_bias, ntype_bias)


def kernel(x, edge_index, weight, conv_bias, ntype_bias):
    ones = jnp.ones((N, DP - HD), x.dtype)
    xp = jnp.stack([jnp.concatenate([x[:, :HD], ones], axis=1),
                    jnp.concatenate([x[:, HD:], ones], axis=1)])
    src = edge_index[:, 0, :].reshape(R, NSUB, NCH, CH)
    dst = edge_index[:, 1, :].reshape(R, NSUB, NCH, CH)
    zeros = jnp.zeros((SLAB, DP), jnp.float32)
    agg = _sc_aggregate(xp, src, dst, zeros)
    return _tc_finish(agg, weight, conv_bias, ntype_bias)
